# baseline (speedup 1.0000x reference)
import functools

import jax
import jax.numpy as jnp
from jax import lax
from jax.experimental import pallas as pl
from jax.experimental.pallas import tpu as pltpu
from jax.experimental.pallas import tpu_sc as plsc

N_FEATS = 4
C = 96
H = 384
W = 384
HW = H * W
NUM_PATCHES = 4096

NC = 2
NS = 16
NW = NC * NS
B_PER_W = NUM_PATCHES // NW
CT = C // 8
LG = B_PER_W // 16


def _sc_gather_normalize(feats_flat, pid):
    mesh = plsc.VectorSubcoreMesh(
        core_axis_name="c", subcore_axis_name="s", num_cores=NC,
        num_subcores=NS)

    @functools.partial(
        pl.kernel,
        out_type=jax.ShapeDtypeStruct((N_FEATS, CT, NW, 8, 128),
                                      jnp.float32),
        mesh=mesh,
        scratch_types=[
            pltpu.VMEM((N_FEATS, B_PER_W), jnp.int32),
            pltpu.VMEM((B_PER_W,), jnp.int32),
            pltpu.VMEM((B_PER_W,), jnp.float32),
            pltpu.VMEM((C, B_PER_W), jnp.int32),
            pltpu.VMEM((2, CT, 8, B_PER_W), jnp.float32),
            pltpu.SemaphoreType.DMA,
            pltpu.SemaphoreType.DMA,
        ],
    )
    def k(feats_hbm, pid_hbm, out_hbm, pid_v, toff_v, scl_v, idx_v, buf_v,
          gsem, wsem):
        wid = lax.axis_index("s") * NC + lax.axis_index("c")
        base = wid * B_PER_W
        pltpu.sync_copy(pid_hbm.at[:, 0, pl.ds(base, B_PER_W)], pid_v)

        def normalize_and_writeback(g):
            buf = buf_v.at[g % 2]

            for i in range(LG):
                scl_v[pl.ds(i * 16, 16)] = jnp.zeros((16,), jnp.float32)

            def acc_body(c, carry):
                ct = lax.shift_right_logical(c, 3)
                cs = c & 7
                for i in range(LG):
                    v = buf[ct, cs, pl.ds(i * 16, 16)]
                    scl_v[pl.ds(i * 16, 16)] = (
                        scl_v[pl.ds(i * 16, 16)] + v * v)
                return carry

            lax.fori_loop(0, C, acc_body, 0, unroll=False)

            for i in range(LG):
                ss = scl_v[pl.ds(i * 16, 16)]
                bits = lax.bitcast_convert_type(ss, jnp.int32)
                seed = 0x5F3759DF - lax.shift_right_logical(bits, 1)
                r = lax.bitcast_convert_type(seed, jnp.float32)
                for _ in range(3):
                    r = r * (1.5 - 0.5 * ss * r * r)
                d = ss * r + 1e-7
                z = r * (2.0 - d * r)
                z = z * (2.0 - d * z)
                scl_v[pl.ds(i * 16, 16)] = z

            def scale_body(c, carry):
                ct = lax.shift_right_logical(c, 3)
                cs = c & 7
                for i in range(LG):
                    buf[ct, cs, pl.ds(i * 16, 16)] = (
                        buf[ct, cs, pl.ds(i * 16, 16)]
                        * scl_v[pl.ds(i * 16, 16)])
                return carry

            lax.fori_loop(0, C, scale_body, 0, unroll=False)

            pltpu.async_copy(buf, out_hbm.at[g, :, wid], wsem)

        for f in range(N_FEATS):
            buf = buf_v.at[f % 2]
            if f >= 2:
                pltpu.make_async_copy(
                    buf, out_hbm.at[f - 2, :, wid], wsem).wait()

            for i in range(LG):
                p = pid_v[f, pl.ds(i * 16, 16)]
                t = lax.shift_right_logical(p, 7)
                h = lax.shift_right_logical(t * 21846, 16)
                tw = t - 3 * h
                ti = lax.shift_right_logical(h, 3) * 3 + tw
                off = (lax.shift_left(ti, 10)
                       + lax.shift_left(h & 7, 7)
                       + (p & 127))
                toff_v[pl.ds(i * 16, 16)] = off

            def build_fire(c, carry):
                pb = (f * C + c) * HW
                for i in range(LG):
                    v = toff_v[pl.ds(i * 16, 16)]
                    idx_v[c, pl.ds(i * 16, 16)] = v + pb
                ct = lax.shift_right_logical(c, 3)
                cs = c & 7
                pltpu.async_copy(
                    feats_hbm.at[idx_v.at[c]], buf.at[ct, cs], gsem)
                return carry

            lax.fori_loop(0, C, build_fire, 0, unroll=False)
            if f >= 1:
                normalize_and_writeback(f - 1)
            pltpu.make_async_copy(out_hbm.at[f, :, wid], buf, gsem).wait()

        normalize_and_writeback(N_FEATS - 1)
        for g in range(N_FEATS - 2, N_FEATS):
            pltpu.make_async_copy(
                buf_v.at[g % 2], out_hbm.at[g, :, wid], wsem).wait()

    return k(feats_flat, pid)


def kernel(feats, patch_ids, num_patches):
    del num_patches
    feats_flat = (feats.reshape(N_FEATS, C, H // 8, 8, W // 128, 128)
                  .transpose(0, 1, 2, 4, 3, 5)
                  .reshape(-1))
    pid = patch_ids.astype(jnp.int32)
    raw = _sc_gather_normalize(feats_flat, pid)
    xt = raw.transpose(0, 1, 3, 2, 4).reshape(N_FEATS, C, NUM_PATCHES)
    return jnp.swapaxes(xt, 1, 2)


# speedup vs baseline: 1.0314x; 1.0314x over previous
import functools

import jax
import jax.numpy as jnp
from jax import lax
from jax.experimental import pallas as pl
from jax.experimental.pallas import tpu as pltpu
from jax.experimental.pallas import tpu_sc as plsc

N_FEATS = 4
C = 96
H = 384
W = 384
HW = H * W
NUM_PATCHES = 4096

NC = 2
NS = 16
NW = NC * NS
B_PER_W = NUM_PATCHES // NW
CT = C // 8
LG = B_PER_W // 16


def _sc_gather_normalize(feats_flat, pid):
    mesh = plsc.VectorSubcoreMesh(
        core_axis_name="c", subcore_axis_name="s", num_cores=NC,
        num_subcores=NS)

    @functools.partial(
        pl.kernel,
        out_type=jax.ShapeDtypeStruct((N_FEATS, CT, NW, 8, 128),
                                      jnp.float32),
        mesh=mesh,
        scratch_types=[
            pltpu.VMEM((N_FEATS, B_PER_W), jnp.int32),
            pltpu.VMEM((B_PER_W,), jnp.int32),
            pltpu.VMEM((B_PER_W,), jnp.float32),
            pltpu.VMEM((C, B_PER_W), jnp.int32),
            pltpu.VMEM((2, CT, 8, B_PER_W), jnp.float32),
            pltpu.SemaphoreType.DMA,
            pltpu.SemaphoreType.DMA,
        ],
    )
    def k(feats_hbm, pid_hbm, out_hbm, pid_v, toff_v, scl_v, idx_v, buf_v,
          gsem, wsem):
        wid = lax.axis_index("s") * NC + lax.axis_index("c")
        base = wid * B_PER_W
        pltpu.sync_copy(pid_hbm.at[:, 0, pl.ds(base, B_PER_W)], pid_v)

        def newton_scales():
            for i in range(LG):
                ss = scl_v[pl.ds(i * 16, 16)]
                bits = lax.bitcast_convert_type(ss, jnp.int32)
                seed = 0x5F3759DF - lax.shift_right_logical(bits, 1)
                r = lax.bitcast_convert_type(seed, jnp.float32)
                for _ in range(3):
                    r = r * (1.5 - 0.5 * ss * r * r)
                d = ss * r + 1e-7
                z = r * (2.0 - d * r)
                z = z * (2.0 - d * z)
                scl_v[pl.ds(i * 16, 16)] = z

        def scale_and_writeback(g):
            buf = buf_v.at[g % 2]

            def scale_body(c, carry):
                ct = lax.shift_right_logical(c, 3)
                cs = c & 7
                for i in range(LG):
                    buf[ct, cs, pl.ds(i * 16, 16)] = (
                        buf[ct, cs, pl.ds(i * 16, 16)]
                        * scl_v[pl.ds(i * 16, 16)])
                return carry

            lax.fori_loop(0, C, scale_body, 0, unroll=4)
            pltpu.async_copy(buf, out_hbm.at[g, :, wid], wsem)

        for f in range(N_FEATS):
            buf = buf_v.at[f % 2]
            prev = buf_v.at[(f - 1) % 2]
            if f >= 2:
                pltpu.make_async_copy(
                    buf, out_hbm.at[f - 2, :, wid], wsem).wait()

            for i in range(LG):
                p = pid_v[f, pl.ds(i * 16, 16)]
                t = lax.shift_right_logical(p, 7)
                h = lax.shift_right_logical(t * 21846, 16)
                tw = t - 3 * h
                ti = lax.shift_right_logical(h, 3) * 3 + tw
                off = (lax.shift_left(ti, 10)
                       + lax.shift_left(h & 7, 7)
                       + (p & 127))
                toff_v[pl.ds(i * 16, 16)] = off
                if f >= 1:
                    scl_v[pl.ds(i * 16, 16)] = jnp.zeros((16,), jnp.float32)

            def build_fire(c, carry):
                pb = (f * C + c) * HW
                ct = lax.shift_right_logical(c, 3)
                cs = c & 7
                for i in range(LG):
                    v = toff_v[pl.ds(i * 16, 16)]
                    idx_v[c, pl.ds(i * 16, 16)] = v + pb
                pltpu.async_copy(
                    feats_hbm.at[idx_v.at[c]], buf.at[ct, cs], gsem)
                if f >= 1:
                    for i in range(LG):
                        v = prev[ct, cs, pl.ds(i * 16, 16)]
                        scl_v[pl.ds(i * 16, 16)] = (
                            scl_v[pl.ds(i * 16, 16)] + v * v)
                return carry

            lax.fori_loop(0, C, build_fire, 0, unroll=2)
            if f >= 1:
                newton_scales()
                scale_and_writeback(f - 1)
            pltpu.make_async_copy(out_hbm.at[f, :, wid], buf, gsem).wait()

        for i in range(LG):
            scl_v[pl.ds(i * 16, 16)] = jnp.zeros((16,), jnp.float32)

        def acc_last(c, carry):
            ct = lax.shift_right_logical(c, 3)
            cs = c & 7
            last = buf_v.at[(N_FEATS - 1) % 2]
            for i in range(LG):
                v = last[ct, cs, pl.ds(i * 16, 16)]
                scl_v[pl.ds(i * 16, 16)] = (
                    scl_v[pl.ds(i * 16, 16)] + v * v)
            return carry

        lax.fori_loop(0, C, acc_last, 0, unroll=4)
        newton_scales()
        scale_and_writeback(N_FEATS - 1)
        for g in range(N_FEATS - 2, N_FEATS):
            pltpu.make_async_copy(
                buf_v.at[g % 2], out_hbm.at[g, :, wid], wsem).wait()

    return k(feats_flat, pid)


def kernel(feats, patch_ids, num_patches):
    del num_patches
    feats_flat = (feats.reshape(N_FEATS, C, H // 8, 8, W // 128, 128)
                  .transpose(0, 1, 2, 4, 3, 5)
                  .reshape(-1))
    pid = patch_ids.astype(jnp.int32)
    raw = _sc_gather_normalize(feats_flat, pid)
    xt = raw.transpose(0, 1, 3, 2, 4).reshape(N_FEATS, C, NUM_PATCHES)
    return jnp.swapaxes(xt, 1, 2)


# speedup vs baseline: 1.1111x; 1.0773x over previous
import functools

import jax
import jax.numpy as jnp
from jax import lax
from jax.experimental import pallas as pl
from jax.experimental.pallas import tpu as pltpu
from jax.experimental.pallas import tpu_sc as plsc

N_FEATS = 4
C = 96
H = 384
W = 384
HW = H * W
NUM_PATCHES = 4096

NC = 2
NS = 16
NW = NC * NS
B_PER_W = NUM_PATCHES // NW
CT = C // 8
LG = B_PER_W // 16


def _sc_gather(feats_flat, pid):
    mesh = plsc.VectorSubcoreMesh(
        core_axis_name="c", subcore_axis_name="s", num_cores=NC,
        num_subcores=NS)

    @functools.partial(
        pl.kernel,
        out_type=jax.ShapeDtypeStruct((N_FEATS, CT, NW, 8, 128),
                                      jnp.float32),
        mesh=mesh,
        scratch_types=[
            pltpu.VMEM((N_FEATS, B_PER_W), jnp.int32),
            pltpu.VMEM((B_PER_W,), jnp.int32),
            pltpu.VMEM((C, B_PER_W), jnp.int32),
            pltpu.VMEM((2, CT, 8, B_PER_W), jnp.float32),
            pltpu.SemaphoreType.DMA,
            pltpu.SemaphoreType.DMA,
        ],
    )
    def k(feats_hbm, pid_hbm, out_hbm, pid_v, toff_v, idx_v, buf_v,
          gsem, wsem):
        wid = lax.axis_index("s") * NC + lax.axis_index("c")
        base = wid * B_PER_W
        pltpu.sync_copy(pid_hbm.at[:, 0, pl.ds(base, B_PER_W)], pid_v)

        for f in range(N_FEATS):
            buf = buf_v.at[f % 2]
            if f >= 2:
                pltpu.make_async_copy(
                    buf, out_hbm.at[f - 2, :, wid], wsem).wait()

            for i in range(LG):
                p = pid_v[f, pl.ds(i * 16, 16)]
                t = lax.shift_right_logical(p, 7)
                h = lax.shift_right_logical(t * 21846, 16)
                tw = t - 3 * h
                ti = lax.shift_right_logical(h, 3) * 3 + tw
                off = (lax.shift_left(ti, 10)
                       + lax.shift_left(h & 7, 7)
                       + (p & 127))
                toff_v[pl.ds(i * 16, 16)] = off

            def build_fire(c, carry):
                pb = (f * C + c) * HW
                ct = lax.shift_right_logical(c, 3)
                cs = c & 7
                for i in range(LG):
                    v = toff_v[pl.ds(i * 16, 16)]
                    idx_v[c, pl.ds(i * 16, 16)] = v + pb
                pltpu.async_copy(
                    feats_hbm.at[idx_v.at[c]], buf.at[ct, cs], gsem)
                return carry

            lax.fori_loop(0, C, build_fire, 0, unroll=2)
            if f >= 1:
                pltpu.async_copy(
                    buf_v.at[(f - 1) % 2], out_hbm.at[f - 1, :, wid], wsem)
            pltpu.make_async_copy(out_hbm.at[f, :, wid], buf, gsem).wait()

        pltpu.async_copy(
            buf_v.at[(N_FEATS - 1) % 2],
            out_hbm.at[N_FEATS - 1, :, wid], wsem)
        for g in range(N_FEATS - 2, N_FEATS):
            pltpu.make_async_copy(
                buf_v.at[g % 2], out_hbm.at[g, :, wid], wsem).wait()

    return k(feats_flat, pid)


def _tc_normalize(xt):
    def body(x_ref, o_ref):
        x = x_ref[...]
        ss = jnp.sum(x * x, axis=0, keepdims=True)
        o_ref[...] = x / (jnp.sqrt(ss) + 1e-7)

    return pl.pallas_call(
        body,
        grid=(N_FEATS,),
        in_specs=[pl.BlockSpec((None, C, NUM_PATCHES), lambda i: (i, 0, 0))],
        out_specs=pl.BlockSpec((None, C, NUM_PATCHES), lambda i: (i, 0, 0)),
        out_shape=jax.ShapeDtypeStruct((N_FEATS, C, NUM_PATCHES),
                                       jnp.float32),
    )(xt)


def kernel(feats, patch_ids, num_patches):
    del num_patches
    feats_flat = (feats.reshape(N_FEATS, C, H // 8, 8, W // 128, 128)
                  .transpose(0, 1, 2, 4, 3, 5)
                  .reshape(-1))
    pid = patch_ids.astype(jnp.int32)
    raw = _sc_gather(feats_flat, pid)
    xt = raw.transpose(0, 1, 3, 2, 4).reshape(N_FEATS, C, NUM_PATCHES)
    return jnp.swapaxes(_tc_normalize(xt), 1, 2)


# speedup vs baseline: 1.1382x; 1.0244x over previous
import functools

import jax
import jax.numpy as jnp
from jax import lax
from jax.experimental import pallas as pl
from jax.experimental.pallas import tpu as pltpu
from jax.experimental.pallas import tpu_sc as plsc

N_FEATS = 4
C = 96
H = 384
W = 384
HW = H * W
NUM_PATCHES = 4096

NC = 2
NS = 16
NW = NC * NS
B_PER_W = NUM_PATCHES // NW
CT = C // 8
LG = B_PER_W // 16


def _sc_gather(feats_flat, pid):
    mesh = plsc.VectorSubcoreMesh(
        core_axis_name="c", subcore_axis_name="s", num_cores=NC,
        num_subcores=NS)

    @functools.partial(
        pl.kernel,
        out_type=jax.ShapeDtypeStruct((N_FEATS, CT, NW, 8, 128),
                                      jnp.float32),
        mesh=mesh,
        scratch_types=[
            pltpu.VMEM((N_FEATS, B_PER_W), jnp.int32),
            pltpu.VMEM((B_PER_W,), jnp.int32),
            pltpu.VMEM((2, C, B_PER_W), jnp.int32),
            pltpu.VMEM((2, CT, 8, B_PER_W), jnp.float32),
            pltpu.SemaphoreType.DMA,
            pltpu.SemaphoreType.DMA,
            pltpu.SemaphoreType.DMA,
        ],
    )
    def k(feats_hbm, pid_hbm, out_hbm, pid_v, toff_v, idx_v, buf_v,
          gsem_a, gsem_b, wsem):
        wid = lax.axis_index("s") * NC + lax.axis_index("c")
        base = wid * B_PER_W
        gsems = (gsem_a, gsem_b)
        pltpu.sync_copy(pid_hbm.at[:, 0, pl.ds(base, B_PER_W)], pid_v)

        for f in range(N_FEATS):
            b = f % 2
            buf = buf_v.at[b]
            idx = idx_v.at[b]
            if f >= 2:
                pltpu.make_async_copy(
                    buf, out_hbm.at[f - 2, :, wid], wsem).wait()

            for i in range(LG):
                p = pid_v[f, pl.ds(i * 16, 16)]
                t = lax.shift_right_logical(p, 7)
                h = lax.shift_right_logical(t * 21846, 16)
                tw = t - 3 * h
                ti = lax.shift_right_logical(h, 3) * 3 + tw
                off = (lax.shift_left(ti, 10)
                       + lax.shift_left(h & 7, 7)
                       + (p & 127))
                toff_v[pl.ds(i * 16, 16)] = off

            def build_fire(c, carry):
                pb = (f * C + c) * HW
                ct = lax.shift_right_logical(c, 3)
                cs = c & 7
                for i in range(LG):
                    v = toff_v[pl.ds(i * 16, 16)]
                    idx[c, pl.ds(i * 16, 16)] = v + pb
                pltpu.async_copy(
                    feats_hbm.at[idx.at[c]], buf.at[ct, cs], gsems[b])
                return carry

            lax.fori_loop(0, C, build_fire, 0, unroll=2)
            if f >= 1:
                pltpu.make_async_copy(
                    out_hbm.at[f - 1, :, wid], buf_v.at[1 - b],
                    gsems[1 - b]).wait()
                pltpu.async_copy(
                    buf_v.at[1 - b], out_hbm.at[f - 1, :, wid], wsem)

        last = N_FEATS - 1
        pltpu.make_async_copy(
            out_hbm.at[last, :, wid], buf_v.at[last % 2],
            gsems[last % 2]).wait()
        pltpu.async_copy(
            buf_v.at[last % 2], out_hbm.at[last, :, wid], wsem)
        for g in range(N_FEATS - 2, N_FEATS):
            pltpu.make_async_copy(
                buf_v.at[g % 2], out_hbm.at[g, :, wid], wsem).wait()

    return k(feats_flat, pid)


def _tc_normalize(xt):
    def body(x_ref, o_ref):
        x = x_ref[...]
        ss = jnp.sum(x * x, axis=0, keepdims=True)
        o_ref[...] = x / (jnp.sqrt(ss) + 1e-7)

    return pl.pallas_call(
        body,
        grid=(N_FEATS,),
        in_specs=[pl.BlockSpec((None, C, NUM_PATCHES), lambda i: (i, 0, 0))],
        out_specs=pl.BlockSpec((None, C, NUM_PATCHES), lambda i: (i, 0, 0)),
        out_shape=jax.ShapeDtypeStruct((N_FEATS, C, NUM_PATCHES),
                                       jnp.float32),
    )(xt)


def kernel(feats, patch_ids, num_patches):
    del num_patches
    feats_flat = (feats.reshape(N_FEATS, C, H // 8, 8, W // 128, 128)
                  .transpose(0, 1, 2, 4, 3, 5)
                  .reshape(-1))
    pid = patch_ids.astype(jnp.int32)
    raw = _sc_gather(feats_flat, pid)
    xt = raw.transpose(0, 1, 3, 2, 4).reshape(N_FEATS, C, NUM_PATCHES)
    return jnp.swapaxes(_tc_normalize(xt), 1, 2)
